# baseline (speedup 1.0000x reference)
import jax
import jax.numpy as jnp
from jax import lax
from jax.experimental import pallas as pl
from jax.experimental.pallas import tpu as pltpu
from jax.experimental.pallas import tpu_sc as plsc

_EMBED = 32
_HIDDEN = 128
_OUT = 50
_WINDOW = 5
_BATCH = 16384

_NC = 2
_NS = 16
_NW = _NC * _NS
_TOTAL = _BATCH * _WINDOW
_PER_W = _TOTAL // _NW
_CHUNK = 128
_NCHUNK = _PER_W // _CHUNK

_BB = 1024


def _sc_gather_body(emb_hbm, idx_hbm, out_hbm, idx_v, rows_v, sem):
    wid = lax.axis_index("s") * _NC + lax.axis_index("c")
    pltpu.sync_copy(idx_hbm.at[wid], idx_v)
    copies = []
    for j in range(_NCHUNK):
        copies.append(
            pltpu.async_copy(
                emb_hbm.at[idx_v.at[j]],
                rows_v.at[pl.ds(j * _CHUNK, _CHUNK)],
                sem,
            )
        )
    for c in copies:
        c.wait()
    pltpu.sync_copy(rows_v, out_hbm.at[wid])


def _sc_gather(emb, idx3):
    mesh = plsc.VectorSubcoreMesh(core_axis_name="c", subcore_axis_name="s")
    f = pl.kernel(
        _sc_gather_body,
        out_type=jax.ShapeDtypeStruct((_NW, _PER_W, _EMBED), jnp.float32),
        mesh=mesh,
        scratch_types=[
            pltpu.VMEM((_NCHUNK, _CHUNK), jnp.int32),
            pltpu.VMEM((_PER_W, _EMBED), jnp.float32),
            pltpu.SemaphoreType.DMA,
        ],
        compiler_params=pltpu.CompilerParams(use_tc_tiling_on_sc=False),
    )
    return f(emb, idx3)


def _mlp_body(x_ref, w1_ref, b1_ref, w2_ref, b2_ref, o_ref):
    acc = jnp.dot(
        x_ref[0], w1_ref[0], preferred_element_type=jnp.float32
    )
    for k in range(1, _WINDOW):
        acc += jnp.dot(
            x_ref[k], w1_ref[k], preferred_element_type=jnp.float32
        )
    h = jnp.tanh(acc + b1_ref[...])
    o_ref[...] = (
        jnp.dot(h, w2_ref[...], preferred_element_type=jnp.float32) + b2_ref[...]
    )


def _mlp(xg, W1r, b1, W2, b2):
    return pl.pallas_call(
        _mlp_body,
        grid=(_BATCH // _BB,),
        in_specs=[
            pl.BlockSpec((_WINDOW, _BB, _EMBED), lambda i: (0, i, 0)),
            pl.BlockSpec((_WINDOW, _EMBED, _HIDDEN), lambda i: (0, 0, 0)),
            pl.BlockSpec((1, _HIDDEN), lambda i: (0, 0)),
            pl.BlockSpec((_HIDDEN, _OUT), lambda i: (0, 0)),
            pl.BlockSpec((1, _OUT), lambda i: (0, 0)),
        ],
        out_specs=pl.BlockSpec((_BB, _OUT), lambda i: (i, 0)),
        out_shape=jax.ShapeDtypeStruct((_BATCH, _OUT), jnp.float32),
    )(xg, W1r, b1.reshape(1, _HIDDEN), W2, b2.reshape(1, _OUT))


def kernel(x, emb, W1, b1, W2, b2):
    idx3 = x.T.reshape(_NW, _NCHUNK, _CHUNK)
    gathered = _sc_gather(emb, idx3)
    xg = gathered.reshape(_WINDOW, _BATCH, _EMBED)
    W1r = W1.reshape(_WINDOW, _EMBED, _HIDDEN)
    return _mlp(xg, W1r, b1, W2, b2)


# speedup vs baseline: 2.2047x; 2.2047x over previous
import jax
import jax.numpy as jnp
from jax import lax
from jax.experimental import pallas as pl
from jax.experimental.pallas import tpu as pltpu
from jax.experimental.pallas import tpu_sc as plsc

_VOCAB = 1000000
_EMBED = 32
_HIDDEN = 128
_OUT = 50
_WINDOW = 5
_BATCH = 16384

_PACKR = 2048
_PACKB = 4 * _PACKR
_PSTEPS = -(-_VOCAB // _PACKB)
_TABR = _PSTEPS * _PACKR

_NC = 2
_NS = 16
_NW = _NC * _NS
_TOTAL = _BATCH * _WINDOW
_PER_W = _TOTAL // _NW
_CHUNK = 128
_NCHUNK = _PER_W // _CHUNK
_STG = 256
_NSTG = _PER_W // _STG
_CPS = _STG // _CHUNK

_BB = 1024


def _pack_body(t_ref, o_ref):
    t = t_ref[...]
    stacked = jnp.concatenate(
        [t[:, m * _PACKR:(m + 1) * _PACKR] for m in range(4)], axis=0
    )
    o_ref[...] = jnp.swapaxes(stacked, 0, 1)


def _pack(embT):
    return pl.pallas_call(
        _pack_body,
        grid=(_PSTEPS,),
        in_specs=[pl.BlockSpec((_EMBED, _PACKB), lambda i: (0, i))],
        out_specs=pl.BlockSpec((_PACKR, 4 * _EMBED), lambda i: (i, 0)),
        out_shape=jax.ShapeDtypeStruct((_TABR, 4 * _EMBED), jnp.float32),
    )(embT)


def _gather1_body(tab_hbm, idx_hbm, out_hbm, idx_v, buf_v, sem, osem):
    wid = lax.axis_index("s") * _NC + lax.axis_index("c")
    pltpu.sync_copy(idx_hbm.at[wid], idx_v)
    ocopies = [None] * _NSTG
    for s in range(_NSTG):
        if s >= 2:
            ocopies[s - 2].wait()
        gathers = []
        for c in range(_CPS):
            j = s * _CPS + c
            gathers.append(
                pltpu.async_copy(
                    tab_hbm.at[idx_v.at[j]],
                    buf_v.at[s % 2].at[pl.ds(c * _CHUNK, _CHUNK)],
                    sem,
                )
            )
        for g in gathers:
            g.wait()
        ocopies[s] = pltpu.async_copy(
            buf_v.at[s % 2],
            out_hbm.at[pl.ds(wid * _PER_W + s * _STG, _STG)],
            osem,
        )
    ocopies[_NSTG - 2].wait()
    ocopies[_NSTG - 1].wait()


def _gather1(tab, idx3):
    mesh = plsc.VectorSubcoreMesh(core_axis_name="c", subcore_axis_name="s")
    f = pl.kernel(
        _gather1_body,
        out_type=jax.ShapeDtypeStruct((_TOTAL, 4 * _EMBED), jnp.float32),
        mesh=mesh,
        scratch_types=[
            pltpu.VMEM((_NCHUNK, _CHUNK), jnp.int32),
            pltpu.VMEM((2, _STG, 4 * _EMBED), jnp.float32),
            pltpu.SemaphoreType.DMA,
            pltpu.SemaphoreType.DMA,
        ],
        compiler_params=pltpu.CompilerParams(use_tc_tiling_on_sc=False),
    )
    return f(tab, idx3)


def _gather2_body(tab_hbm, idx_hbm, out_hbm, idx_v, rows_v, sem):
    wid = lax.axis_index("s") * _NC + lax.axis_index("c")
    pltpu.sync_copy(idx_hbm.at[wid], idx_v)
    copies = []
    for j in range(_NCHUNK):
        copies.append(
            pltpu.async_copy(
                tab_hbm.at[idx_v.at[j]],
                rows_v.at[pl.ds(j * _CHUNK, _CHUNK)],
                sem,
            )
        )
    for c in copies:
        c.wait()
    pltpu.sync_copy(rows_v, out_hbm.at[pl.ds(wid * _PER_W, _PER_W)])


def _gather2(rows4, idx3):
    mesh = plsc.VectorSubcoreMesh(core_axis_name="c", subcore_axis_name="s")
    f = pl.kernel(
        _gather2_body,
        out_type=jax.ShapeDtypeStruct((_TOTAL, _EMBED), jnp.float32),
        mesh=mesh,
        scratch_types=[
            pltpu.VMEM((_NCHUNK, _CHUNK), jnp.int32),
            pltpu.VMEM((_PER_W, _EMBED), jnp.float32),
            pltpu.SemaphoreType.DMA,
        ],
        compiler_params=pltpu.CompilerParams(use_tc_tiling_on_sc=False),
    )
    return f(rows4.reshape(4 * _TOTAL, _EMBED), idx3)


def _mlp_body(x0, x1, x2, x3, x4, w1_ref, b1_ref, w2_ref, b2_ref, o_ref):
    acc = None
    for k, xk in enumerate((x0, x1, x2, x3, x4)):
        d = jnp.dot(xk[...], w1_ref[k], preferred_element_type=jnp.float32)
        acc = d if acc is None else acc + d
    h = jnp.tanh(acc + b1_ref[...])
    o_ref[...] = (
        jnp.dot(h, w2_ref[...], preferred_element_type=jnp.float32) + b2_ref[...]
    )


def _mlp(xg, W1r, b1, W2, b2):
    xspec = lambda k: pl.BlockSpec(
        (_BB, _EMBED), lambda i, k=k: (k * (_BATCH // _BB) + i, 0)
    )
    return pl.pallas_call(
        _mlp_body,
        grid=(_BATCH // _BB,),
        in_specs=[xspec(k) for k in range(_WINDOW)] + [
            pl.BlockSpec((_WINDOW, _EMBED, _HIDDEN), lambda i: (0, 0, 0)),
            pl.BlockSpec((1, _HIDDEN), lambda i: (0, 0)),
            pl.BlockSpec((_HIDDEN, _OUT), lambda i: (0, 0)),
            pl.BlockSpec((1, _OUT), lambda i: (0, 0)),
        ],
        out_specs=pl.BlockSpec((_BB, _OUT), lambda i: (i, 0)),
        out_shape=jax.ShapeDtypeStruct((_BATCH, _OUT), jnp.float32),
    )(xg, xg, xg, xg, xg, W1r, b1.reshape(1, _HIDDEN), W2, b2.reshape(1, _OUT))


def kernel(x, emb, W1, b1, W2, b2):
    tab = _pack(emb.T)
    xt = x.T
    flat = xt.reshape(_TOTAL)
    row = (flat // _PACKB) * _PACKR + (flat % _PACKR)
    sub = 4 * jnp.arange(_TOTAL, dtype=jnp.int32) + (flat // _PACKR) % 4
    idx1 = row.reshape(_NW, _NCHUNK, _CHUNK)
    idx2 = sub.reshape(_NW, _NCHUNK, _CHUNK)
    rows4 = _gather1(tab, idx1)
    gathered = _gather2(rows4, idx2)
    W1r = W1.reshape(_WINDOW, _EMBED, _HIDDEN)
    return _mlp(gathered, W1r, b1, W2, b2)


# speedup vs baseline: 2.5706x; 1.1660x over previous
import jax
import jax.numpy as jnp
from jax import lax
from jax.experimental import pallas as pl
from jax.experimental.pallas import tpu as pltpu
from jax.experimental.pallas import tpu_sc as plsc

_VOCAB = 1000000
_EMBED = 32
_HIDDEN = 128
_OUT = 50
_WINDOW = 5
_BATCH = 16384

_PACKR = 4096
_PACKB = 4 * _PACKR
_PSTEPS = -(-_VOCAB // _PACKB)
_TABR = _PSTEPS * _PACKR

_NC = 2
_NS = 16
_NW = _NC * _NS
_TOTAL = _BATCH * _WINDOW
_PER_W = _TOTAL // _NW
_CHUNK = 128
_NCHUNK = _PER_W // _CHUNK
_STG = 256
_NSTG = _PER_W // _STG
_CPS = _STG // _CHUNK

_BB = 1024


def _pack_body(t_ref, o_ref):
    t = t_ref[...]
    stacked = jnp.concatenate(
        [t[:, m * _PACKR:(m + 1) * _PACKR] for m in range(4)], axis=0
    )
    o_ref[...] = jnp.swapaxes(stacked, 0, 1)


def _pack(embT):
    return pl.pallas_call(
        _pack_body,
        grid=(_PSTEPS,),
        in_specs=[pl.BlockSpec((_EMBED, _PACKB), lambda i: (0, i))],
        out_specs=pl.BlockSpec((_PACKR, 4 * _EMBED), lambda i: (i, 0)),
        out_shape=jax.ShapeDtypeStruct((_TABR, 4 * _EMBED), jnp.float32),
    )(embT)


def _gather1_body(tab_hbm, idx_hbm, out_hbm, idx_v, buf_v, sem, osem):
    wid = lax.axis_index("s") * _NC + lax.axis_index("c")
    pltpu.sync_copy(idx_hbm.at[wid], idx_v)
    ocopies = [None] * _NSTG
    for s in range(_NSTG):
        if s >= 2:
            ocopies[s - 2].wait()
        gathers = []
        for c in range(_CPS):
            j = s * _CPS + c
            gathers.append(
                pltpu.async_copy(
                    tab_hbm.at[idx_v.at[j]],
                    buf_v.at[s % 2].at[pl.ds(c * _CHUNK, _CHUNK)],
                    sem,
                )
            )
        for g in gathers:
            g.wait()
        ocopies[s] = pltpu.async_copy(
            buf_v.at[s % 2],
            out_hbm.at[pl.ds(wid * _PER_W + s * _STG, _STG)],
            osem,
        )
    ocopies[_NSTG - 2].wait()
    ocopies[_NSTG - 1].wait()


def _gather1(tab, idx3):
    mesh = plsc.VectorSubcoreMesh(core_axis_name="c", subcore_axis_name="s")
    f = pl.kernel(
        _gather1_body,
        out_type=jax.ShapeDtypeStruct((_TOTAL, 4 * _EMBED), jnp.float32),
        mesh=mesh,
        scratch_types=[
            pltpu.VMEM((_NCHUNK, _CHUNK), jnp.int32),
            pltpu.VMEM((2, _STG, 4 * _EMBED), jnp.float32),
            pltpu.SemaphoreType.DMA,
            pltpu.SemaphoreType.DMA,
        ],
        compiler_params=pltpu.CompilerParams(use_tc_tiling_on_sc=False),
    )
    return f(tab, idx3)


def _gather2_body(tab_hbm, idx_hbm, out_hbm, idx_v, rows_v, sem):
    wid = lax.axis_index("s") * _NC + lax.axis_index("c")
    pltpu.sync_copy(idx_hbm.at[wid], idx_v)
    copies = []
    for j in range(_NCHUNK):
        copies.append(
            pltpu.async_copy(
                tab_hbm.at[idx_v.at[j]],
                rows_v.at[pl.ds(j * _CHUNK, _CHUNK)],
                sem,
            )
        )
    for c in copies:
        c.wait()
    pltpu.sync_copy(rows_v, out_hbm.at[pl.ds(wid * _PER_W, _PER_W)])


def _gather2(rows4, idx3):
    mesh = plsc.VectorSubcoreMesh(core_axis_name="c", subcore_axis_name="s")
    f = pl.kernel(
        _gather2_body,
        out_type=jax.ShapeDtypeStruct((_TOTAL, _EMBED), jnp.float32),
        mesh=mesh,
        scratch_types=[
            pltpu.VMEM((_NCHUNK, _CHUNK), jnp.int32),
            pltpu.VMEM((_PER_W, _EMBED), jnp.float32),
            pltpu.SemaphoreType.DMA,
        ],
        compiler_params=pltpu.CompilerParams(use_tc_tiling_on_sc=False),
    )
    return f(rows4.reshape(4 * _TOTAL, _EMBED), idx3)


def _mlp_body(x0, x1, x2, x3, x4, w1_ref, b1_ref, w2_ref, b2_ref, o_ref):
    acc = None
    for k, xk in enumerate((x0, x1, x2, x3, x4)):
        d = jnp.dot(xk[...], w1_ref[k], preferred_element_type=jnp.float32)
        acc = d if acc is None else acc + d
    h = jnp.tanh(acc + b1_ref[...])
    o_ref[...] = (
        jnp.dot(h, w2_ref[...], preferred_element_type=jnp.float32) + b2_ref[...]
    )


def _mlp(xg, W1r, b1, W2, b2):
    xspec = lambda k: pl.BlockSpec(
        (_BB, _EMBED), lambda i, k=k: (k * (_BATCH // _BB) + i, 0)
    )
    return pl.pallas_call(
        _mlp_body,
        grid=(_BATCH // _BB,),
        in_specs=[xspec(k) for k in range(_WINDOW)] + [
            pl.BlockSpec((_WINDOW, _EMBED, _HIDDEN), lambda i: (0, 0, 0)),
            pl.BlockSpec((1, _HIDDEN), lambda i: (0, 0)),
            pl.BlockSpec((_HIDDEN, _OUT), lambda i: (0, 0)),
            pl.BlockSpec((1, _OUT), lambda i: (0, 0)),
        ],
        out_specs=pl.BlockSpec((_BB, _OUT), lambda i: (i, 0)),
        out_shape=jax.ShapeDtypeStruct((_BATCH, _OUT), jnp.float32),
    )(xg, xg, xg, xg, xg, W1r, b1.reshape(1, _HIDDEN), W2, b2.reshape(1, _OUT))


def kernel(x, emb, W1, b1, W2, b2):
    tab = _pack(emb.T)
    xt = x.T
    flat = xt.reshape(_TOTAL)
    row = (flat // _PACKB) * _PACKR + (flat % _PACKR)
    sub = 4 * jnp.arange(_TOTAL, dtype=jnp.int32) + (flat // _PACKR) % 4
    idx1 = row.reshape(_NW, _NCHUNK, _CHUNK)
    idx2 = sub.reshape(_NW, _NCHUNK, _CHUNK)
    rows4 = _gather1(tab, idx1)
    gathered = _gather2(rows4, idx2)
    W1r = W1.reshape(_WINDOW, _EMBED, _HIDDEN)
    return _mlp(gathered, W1r, b1, W2, b2)


# speedup vs baseline: 2.7560x; 1.0721x over previous
import jax
import jax.numpy as jnp
from jax import lax
from jax.experimental import pallas as pl
from jax.experimental.pallas import tpu as pltpu
from jax.experimental.pallas import tpu_sc as plsc

_VOCAB = 1000000
_EMBED = 32
_HIDDEN = 128
_OUT = 50
_WINDOW = 5
_BATCH = 16384

_PACKR = 4096
_PACKB = 4 * _PACKR
_PSTEPS = -(-_VOCAB // _PACKB)
_TABR = _PSTEPS * _PACKR

_NC = 2
_NS = 16
_NW = _NC * _NS
_TOTAL = _BATCH * _WINDOW
_PER_W = _TOTAL // _NW
_CHUNK = 128
_NCHUNK = _PER_W // _CHUNK
_STG = 256
_NSTG = _PER_W // _STG
_CPS = _STG // _CHUNK

_BB = 1024


def _pack_body(t_ref, o_ref):
    t = t_ref[...]
    stacked = jnp.concatenate(
        [t[:, m * _PACKR:(m + 1) * _PACKR] for m in range(4)], axis=0
    )
    o_ref[...] = jnp.swapaxes(stacked, 0, 1)


def _pack(embT):
    return pl.pallas_call(
        _pack_body,
        grid=(_PSTEPS,),
        in_specs=[pl.BlockSpec((_EMBED, _PACKB), lambda i: (0, i))],
        out_specs=pl.BlockSpec((_PACKR, 4 * _EMBED), lambda i: (i, 0)),
        out_shape=jax.ShapeDtypeStruct((_TABR, 4 * _EMBED), jnp.float32),
    )(embT)


def _gather1_body(tab_hbm, idx_hbm, out_hbm, idx_v, buf_v, sem, osem):
    wid = lax.axis_index("s") * _NC + lax.axis_index("c")
    pltpu.sync_copy(idx_hbm.at[wid], idx_v)
    ocopies = [None] * _NSTG
    for s in range(_NSTG):
        if s >= 2:
            ocopies[s - 2].wait()
        gathers = []
        for c in range(_CPS):
            j = s * _CPS + c
            gathers.append(
                pltpu.async_copy(
                    tab_hbm.at[idx_v.at[j]],
                    buf_v.at[s % 2].at[pl.ds(c * _CHUNK, _CHUNK)],
                    sem,
                )
            )
        for g in gathers:
            g.wait()
        ocopies[s] = pltpu.async_copy(
            buf_v.at[s % 2],
            out_hbm.at[pl.ds(wid * _PER_W + s * _STG, _STG)],
            osem,
        )
    ocopies[_NSTG - 2].wait()
    ocopies[_NSTG - 1].wait()


def _gather1(tab, idx3):
    mesh = plsc.VectorSubcoreMesh(core_axis_name="c", subcore_axis_name="s")
    f = pl.kernel(
        _gather1_body,
        out_type=jax.ShapeDtypeStruct((_TOTAL, 4 * _EMBED), jnp.float32),
        mesh=mesh,
        scratch_types=[
            pltpu.VMEM((_NCHUNK, _CHUNK), jnp.int32),
            pltpu.VMEM((2, _STG, 4 * _EMBED), jnp.float32),
            pltpu.SemaphoreType.DMA,
            pltpu.SemaphoreType.DMA,
        ],
        compiler_params=pltpu.CompilerParams(use_tc_tiling_on_sc=False),
    )
    return f(tab, idx3)


def _gather2_body(tab_hbm, idx_hbm, out_hbm, idx_v, rows_v, sem):
    wid = lax.axis_index("s") * _NC + lax.axis_index("c")
    pltpu.sync_copy(idx_hbm.at[wid], idx_v)
    copies = []
    for j in range(_NCHUNK):
        copies.append(
            pltpu.async_copy(
                tab_hbm.at[idx_v.at[j]],
                rows_v.at[pl.ds(j * _CHUNK, _CHUNK)],
                sem,
            )
        )
    for c in copies:
        c.wait()
    pltpu.sync_copy(rows_v, out_hbm.at[pl.ds(wid * _PER_W, _PER_W)])


def _gather2(rows4, idx3):
    mesh = plsc.VectorSubcoreMesh(core_axis_name="c", subcore_axis_name="s")
    f = pl.kernel(
        _gather2_body,
        out_type=jax.ShapeDtypeStruct((_TOTAL, _EMBED), jnp.float32),
        mesh=mesh,
        scratch_types=[
            pltpu.VMEM((_NCHUNK, _CHUNK), jnp.int32),
            pltpu.VMEM((_PER_W, _EMBED), jnp.float32),
            pltpu.SemaphoreType.DMA,
        ],
        compiler_params=pltpu.CompilerParams(use_tc_tiling_on_sc=False),
    )
    return f(rows4.reshape(4 * _TOTAL, _EMBED), idx3)


_RPB = _BB * _EMBED // 128


def _mlp_body(x0, x1, x2, x3, x4, w1_ref, b1_ref, w2_ref, b2_ref, o_ref):
    for s in range(4):
        acc = None
        for k, xk in enumerate((x0, x1, x2, x3, x4)):
            d = jnp.dot(
                xk[:, s * _EMBED:(s + 1) * _EMBED],
                w1_ref[k],
                preferred_element_type=jnp.float32,
            )
            acc = d if acc is None else acc + d
        h = jnp.tanh(acc + b1_ref[...])
        o_ref[s] = (
            jnp.dot(h, w2_ref[...], preferred_element_type=jnp.float32)
            + b2_ref[...]
        )


def _mlp(xg, W1r, b1, W2, b2):
    xspec = lambda k: pl.BlockSpec(
        (_RPB, 128), lambda i, k=k: (k * (_BATCH // _BB) + i, 0)
    )
    out4 = pl.pallas_call(
        _mlp_body,
        grid=(_BATCH // _BB,),
        in_specs=[xspec(k) for k in range(_WINDOW)] + [
            pl.BlockSpec((_WINDOW, _EMBED, _HIDDEN), lambda i: (0, 0, 0)),
            pl.BlockSpec((1, _HIDDEN), lambda i: (0, 0)),
            pl.BlockSpec((_HIDDEN, _OUT), lambda i: (0, 0)),
            pl.BlockSpec((1, _OUT), lambda i: (0, 0)),
        ],
        out_specs=pl.BlockSpec((4, _RPB, _OUT), lambda i: (0, i, 0)),
        out_shape=jax.ShapeDtypeStruct((4, _BATCH // 4, _OUT), jnp.float32),
    )(xg, xg, xg, xg, xg, W1r, b1.reshape(1, _HIDDEN), W2, b2.reshape(1, _OUT))
    return jnp.transpose(out4, (1, 0, 2)).reshape(_BATCH, _OUT)


def kernel(x, emb, W1, b1, W2, b2):
    tab = _pack(emb.T)
    xt = x.T
    flat = xt.reshape(_TOTAL)
    row = (flat // _PACKB) * _PACKR + (flat % _PACKR)
    sub = 4 * jnp.arange(_TOTAL, dtype=jnp.int32) + (flat // _PACKR) % 4
    idx1 = row.reshape(_NW, _NCHUNK, _CHUNK)
    idx2 = sub.reshape(_NW, _NCHUNK, _CHUNK)
    rows4 = _gather1(tab, idx1)
    gathered = _gather2(rows4, idx2)
    graw = gathered.reshape(_TOTAL * _EMBED // 128, 128)
    W1r = W1.reshape(_WINDOW, _EMBED, _HIDDEN)
    return _mlp(graw, W1r, b1, W2, b2)


# speedup vs baseline: 2.9497x; 1.0703x over previous
import jax
import jax.numpy as jnp
from jax import lax
from jax.experimental import pallas as pl
from jax.experimental.pallas import tpu as pltpu
from jax.experimental.pallas import tpu_sc as plsc

_VOCAB = 1000000
_EMBED = 32
_HIDDEN = 128
_OUT = 50
_WINDOW = 5
_BATCH = 16384

_PACKR = 8192
_PACKB = 4 * _PACKR
_PSTEPS = -(-_VOCAB // _PACKB)
_TABR = _PSTEPS * _PACKR

_NC = 2
_NS = 16
_NW = _NC * _NS
_TOTAL = _BATCH * _WINDOW
_PER_W = _TOTAL // _NW
_CHUNK = 128
_NCHUNK = _PER_W // _CHUNK
_STG = 256
_NSTG = _PER_W // _STG
_CPS = _STG // _CHUNK

_BB = 1024


def _pack_body(t_ref, o_ref):
    t = t_ref[...]
    stacked = jnp.concatenate(
        [t[:, m * _PACKR:(m + 1) * _PACKR] for m in range(4)], axis=0
    )
    o_ref[...] = jnp.swapaxes(stacked, 0, 1)


def _pack(embT):
    return pl.pallas_call(
        _pack_body,
        grid=(_PSTEPS,),
        in_specs=[pl.BlockSpec((_EMBED, _PACKB), lambda i: (0, i))],
        out_specs=pl.BlockSpec((_PACKR, 4 * _EMBED), lambda i: (i, 0)),
        out_shape=jax.ShapeDtypeStruct((_TABR, 4 * _EMBED), jnp.float32),
    )(embT)


def _gather1_body(tab_hbm, idx_hbm, out_hbm, idx_v, buf_v, sem, osem):
    wid = lax.axis_index("s") * _NC + lax.axis_index("c")
    pltpu.sync_copy(idx_hbm.at[wid], idx_v)
    ocopies = [None] * _NSTG
    for s in range(_NSTG):
        if s >= 2:
            ocopies[s - 2].wait()
        gathers = []
        for c in range(_CPS):
            j = s * _CPS + c
            gathers.append(
                pltpu.async_copy(
                    tab_hbm.at[idx_v.at[j]],
                    buf_v.at[s % 2].at[pl.ds(c * _CHUNK, _CHUNK)],
                    sem,
                )
            )
        for g in gathers:
            g.wait()
        ocopies[s] = pltpu.async_copy(
            buf_v.at[s % 2],
            out_hbm.at[pl.ds(wid * _PER_W + s * _STG, _STG)],
            osem,
        )
    ocopies[_NSTG - 2].wait()
    ocopies[_NSTG - 1].wait()


def _gather1(tab, idx3):
    mesh = plsc.VectorSubcoreMesh(core_axis_name="c", subcore_axis_name="s")
    f = pl.kernel(
        _gather1_body,
        out_type=jax.ShapeDtypeStruct((_TOTAL, 4 * _EMBED), jnp.float32),
        mesh=mesh,
        scratch_types=[
            pltpu.VMEM((_NCHUNK, _CHUNK), jnp.int32),
            pltpu.VMEM((2, _STG, 4 * _EMBED), jnp.float32),
            pltpu.SemaphoreType.DMA,
            pltpu.SemaphoreType.DMA,
        ],
        compiler_params=pltpu.CompilerParams(use_tc_tiling_on_sc=False),
    )
    return f(tab, idx3)


def _gather2_body(tab_hbm, idx_hbm, out_hbm, idx_v, rows_v, sem):
    wid = lax.axis_index("s") * _NC + lax.axis_index("c")
    pltpu.sync_copy(idx_hbm.at[wid], idx_v)
    copies = []
    for j in range(_NCHUNK):
        copies.append(
            pltpu.async_copy(
                tab_hbm.at[idx_v.at[j]],
                rows_v.at[pl.ds(j * _CHUNK, _CHUNK)],
                sem,
            )
        )
    for c in copies:
        c.wait()
    pltpu.sync_copy(rows_v, out_hbm.at[pl.ds(wid * _PER_W, _PER_W)])


def _gather2(rows4, idx3):
    mesh = plsc.VectorSubcoreMesh(core_axis_name="c", subcore_axis_name="s")
    f = pl.kernel(
        _gather2_body,
        out_type=jax.ShapeDtypeStruct((_TOTAL, _EMBED), jnp.float32),
        mesh=mesh,
        scratch_types=[
            pltpu.VMEM((_NCHUNK, _CHUNK), jnp.int32),
            pltpu.VMEM((_PER_W, _EMBED), jnp.float32),
            pltpu.SemaphoreType.DMA,
        ],
        compiler_params=pltpu.CompilerParams(use_tc_tiling_on_sc=False),
    )
    return f(rows4.reshape(4 * _TOTAL, _EMBED), idx3)


_RPB = _BB * _EMBED // 128


def _mlp_body(x0, x1, x2, x3, x4, w1_ref, b1_ref, w2_ref, b2_ref, o_ref):
    for s in range(4):
        acc = None
        for k, xk in enumerate((x0, x1, x2, x3, x4)):
            d = jnp.dot(
                xk[:, s * _EMBED:(s + 1) * _EMBED],
                w1_ref[k],
                preferred_element_type=jnp.float32,
            )
            acc = d if acc is None else acc + d
        h = jnp.tanh(acc + b1_ref[...])
        o_ref[s] = (
            jnp.dot(h, w2_ref[...], preferred_element_type=jnp.float32)
            + b2_ref[...]
        )


def _mlp(xg, W1r, b1, W2, b2):
    xspec = lambda k: pl.BlockSpec(
        (_RPB, 128), lambda i, k=k: (k * (_BATCH // _BB) + i, 0)
    )
    out4 = pl.pallas_call(
        _mlp_body,
        grid=(_BATCH // _BB,),
        in_specs=[xspec(k) for k in range(_WINDOW)] + [
            pl.BlockSpec((_WINDOW, _EMBED, _HIDDEN), lambda i: (0, 0, 0)),
            pl.BlockSpec((1, _HIDDEN), lambda i: (0, 0)),
            pl.BlockSpec((_HIDDEN, _OUT), lambda i: (0, 0)),
            pl.BlockSpec((1, _OUT), lambda i: (0, 0)),
        ],
        out_specs=pl.BlockSpec((4, _RPB, _OUT), lambda i: (0, i, 0)),
        out_shape=jax.ShapeDtypeStruct((4, _BATCH // 4, _OUT), jnp.float32),
    )(xg, xg, xg, xg, xg, W1r, b1.reshape(1, _HIDDEN), W2, b2.reshape(1, _OUT))
    return jnp.transpose(out4, (1, 0, 2)).reshape(_BATCH, _OUT)


def kernel(x, emb, W1, b1, W2, b2):
    tab = _pack(emb.T)
    xt = x.T
    flat = xt.reshape(_TOTAL)
    row = (flat // _PACKB) * _PACKR + (flat % _PACKR)
    sub = 4 * jnp.arange(_TOTAL, dtype=jnp.int32) + (flat // _PACKR) % 4
    idx1 = row.reshape(_NW, _NCHUNK, _CHUNK)
    idx2 = sub.reshape(_NW, _NCHUNK, _CHUNK)
    rows4 = _gather1(tab, idx1)
    gathered = _gather2(rows4, idx2)
    graw = gathered.reshape(_TOTAL * _EMBED // 128, 128)
    W1r = W1.reshape(_WINDOW, _EMBED, _HIDDEN)
    return _mlp(graw, W1r, b1, W2, b2)


# speedup vs baseline: 2.9538x; 1.0014x over previous
import jax
import jax.numpy as jnp
from jax import lax
from jax.experimental import pallas as pl
from jax.experimental.pallas import tpu as pltpu
from jax.experimental.pallas import tpu_sc as plsc

_VOCAB = 1000000
_EMBED = 32
_HIDDEN = 128
_OUT = 50
_WINDOW = 5
_BATCH = 16384

_PACKR = 16384
_PACKB = 4 * _PACKR
_PSTEPS = -(-_VOCAB // _PACKB)
_TABR = _PSTEPS * _PACKR

_NC = 2
_NS = 16
_NW = _NC * _NS
_TOTAL = _BATCH * _WINDOW
_PER_W = _TOTAL // _NW
_CHUNK = 128
_NCHUNK = _PER_W // _CHUNK
_STG = 256
_NSTG = _PER_W // _STG
_CPS = _STG // _CHUNK

_BB = 1024


def _pack_body(t_ref, o_ref):
    t = t_ref[...]
    stacked = jnp.concatenate(
        [t[:, m * _PACKR:(m + 1) * _PACKR] for m in range(4)], axis=0
    )
    o_ref[...] = jnp.swapaxes(stacked, 0, 1)


def _pack(embT):
    return pl.pallas_call(
        _pack_body,
        grid=(_PSTEPS,),
        in_specs=[pl.BlockSpec((_EMBED, _PACKB), lambda i: (0, i))],
        out_specs=pl.BlockSpec((_PACKR, 4 * _EMBED), lambda i: (i, 0)),
        out_shape=jax.ShapeDtypeStruct((_TABR, 4 * _EMBED), jnp.float32),
    )(embT)


def _gather1_body(tab_hbm, idx_hbm, out_hbm, idx_v, buf_v, sem, osem):
    wid = lax.axis_index("s") * _NC + lax.axis_index("c")
    pltpu.sync_copy(idx_hbm.at[wid], idx_v)
    ocopies = [None] * _NSTG
    for s in range(_NSTG):
        if s >= 2:
            ocopies[s - 2].wait()
        gathers = []
        for c in range(_CPS):
            j = s * _CPS + c
            gathers.append(
                pltpu.async_copy(
                    tab_hbm.at[idx_v.at[j]],
                    buf_v.at[s % 2].at[pl.ds(c * _CHUNK, _CHUNK)],
                    sem,
                )
            )
        for g in gathers:
            g.wait()
        ocopies[s] = pltpu.async_copy(
            buf_v.at[s % 2],
            out_hbm.at[pl.ds(wid * _PER_W + s * _STG, _STG)],
            osem,
        )
    ocopies[_NSTG - 2].wait()
    ocopies[_NSTG - 1].wait()


def _gather1(tab, idx3):
    mesh = plsc.VectorSubcoreMesh(core_axis_name="c", subcore_axis_name="s")
    f = pl.kernel(
        _gather1_body,
        out_type=jax.ShapeDtypeStruct((_TOTAL, 4 * _EMBED), jnp.float32),
        mesh=mesh,
        scratch_types=[
            pltpu.VMEM((_NCHUNK, _CHUNK), jnp.int32),
            pltpu.VMEM((2, _STG, 4 * _EMBED), jnp.float32),
            pltpu.SemaphoreType.DMA,
            pltpu.SemaphoreType.DMA,
        ],
        compiler_params=pltpu.CompilerParams(use_tc_tiling_on_sc=False),
    )
    return f(tab, idx3)


def _gather2_body(tab_hbm, idx_hbm, out_hbm, idx_v, rows_v, sem):
    wid = lax.axis_index("s") * _NC + lax.axis_index("c")
    pltpu.sync_copy(idx_hbm.at[wid], idx_v)
    copies = []
    for j in range(_NCHUNK):
        copies.append(
            pltpu.async_copy(
                tab_hbm.at[idx_v.at[j]],
                rows_v.at[pl.ds(j * _CHUNK, _CHUNK)],
                sem,
            )
        )
    for c in copies:
        c.wait()
    pltpu.sync_copy(rows_v, out_hbm.at[pl.ds(wid * _PER_W, _PER_W)])


def _gather2(rows4, idx3):
    mesh = plsc.VectorSubcoreMesh(core_axis_name="c", subcore_axis_name="s")
    f = pl.kernel(
        _gather2_body,
        out_type=jax.ShapeDtypeStruct((_TOTAL, _EMBED), jnp.float32),
        mesh=mesh,
        scratch_types=[
            pltpu.VMEM((_NCHUNK, _CHUNK), jnp.int32),
            pltpu.VMEM((_PER_W, _EMBED), jnp.float32),
            pltpu.SemaphoreType.DMA,
        ],
        compiler_params=pltpu.CompilerParams(use_tc_tiling_on_sc=False),
    )
    return f(rows4.reshape(4 * _TOTAL, _EMBED), idx3)


_RPB = _BB * _EMBED // 128


def _mlp_body(x0, x1, x2, x3, x4, w1_ref, b1_ref, w2_ref, b2_ref, o_ref):
    for s in range(4):
        acc = None
        for k, xk in enumerate((x0, x1, x2, x3, x4)):
            d = jnp.dot(
                xk[:, s * _EMBED:(s + 1) * _EMBED],
                w1_ref[k],
                preferred_element_type=jnp.float32,
            )
            acc = d if acc is None else acc + d
        h = jnp.tanh(acc + b1_ref[...])
        o_ref[s] = (
            jnp.dot(h, w2_ref[...], preferred_element_type=jnp.float32)
            + b2_ref[...]
        )


def _mlp(xg, W1r, b1, W2, b2):
    xspec = lambda k: pl.BlockSpec(
        (_RPB, 128), lambda i, k=k: (k * (_BATCH // _BB) + i, 0)
    )
    out4 = pl.pallas_call(
        _mlp_body,
        grid=(_BATCH // _BB,),
        in_specs=[xspec(k) for k in range(_WINDOW)] + [
            pl.BlockSpec((_WINDOW, _EMBED, _HIDDEN), lambda i: (0, 0, 0)),
            pl.BlockSpec((1, _HIDDEN), lambda i: (0, 0)),
            pl.BlockSpec((_HIDDEN, _OUT), lambda i: (0, 0)),
            pl.BlockSpec((1, _OUT), lambda i: (0, 0)),
        ],
        out_specs=pl.BlockSpec((4, _RPB, _OUT), lambda i: (0, i, 0)),
        out_shape=jax.ShapeDtypeStruct((4, _BATCH // 4, _OUT), jnp.float32),
    )(xg, xg, xg, xg, xg, W1r, b1.reshape(1, _HIDDEN), W2, b2.reshape(1, _OUT))
    return jnp.transpose(out4, (1, 0, 2)).reshape(_BATCH, _OUT)


def kernel(x, emb, W1, b1, W2, b2):
    tab = _pack(emb.T)
    xt = x.T
    flat = xt.reshape(_TOTAL)
    row = (flat // _PACKB) * _PACKR + (flat % _PACKR)
    sub = 4 * jnp.arange(_TOTAL, dtype=jnp.int32) + (flat // _PACKR) % 4
    idx1 = row.reshape(_NW, _NCHUNK, _CHUNK)
    idx2 = sub.reshape(_NW, _NCHUNK, _CHUNK)
    rows4 = _gather1(tab, idx1)
    gathered = _gather2(rows4, idx2)
    graw = gathered.reshape(_TOTAL * _EMBED // 128, 128)
    W1r = W1.reshape(_WINDOW, _EMBED, _HIDDEN)
    return _mlp(graw, W1r, b1, W2, b2)
